# baseline (speedup 1.0000x reference)
import functools

import jax
import jax.numpy as jnp
from jax import lax
from jax.experimental import pallas as pl
from jax.experimental.pallas import tpu as pltpu
from jax.experimental.pallas import tpu_sc as plsc

N = 10000
F = 128
E = 320000

NC = 2
NS = 16
NW = NC * NS
EPW = E // NW
K = 96
NCHUNK = 105
EPT = K * NCHUNK
ROW_STRIDE = 624
ROW_COPY = 640


def _spmm_partials(h, packed, vals3, zeros):
    mesh = plsc.VectorSubcoreMesh(core_axis_name="c", subcore_axis_name="s")

    @functools.partial(
        pl.kernel,
        out_type=jax.ShapeDtypeStruct((NC, N, F), jnp.float32),
        mesh=mesh,
        scratch_types=[
            pltpu.VMEM((4, 2, K), jnp.int32),
            pltpu.VMEM((4, K), jnp.float32),
            pltpu.VMEM((K, F), jnp.float32),
            pltpu.VMEM((K, F), jnp.float32),
            pltpu.VMEM_SHARED((N, F), jnp.float32),
            [pltpu.SemaphoreType.DMA] * 4,
            [pltpu.SemaphoreType.DMA] * 2,
            [pltpu.SemaphoreType.DMA] * 2,
        ],
    )
    def k(h_hbm, e_hbm, v_hbm, z_hbm, out_hbm,
          pbuf, vbuf, rows0_v, rows1_v, acc_sh, isems, gsems, ssems):
        cid = lax.axis_index("c")
        sid = lax.axis_index("s")
        wid = cid * NS + sid
        rows = (rows0_v, rows1_v)

        rstart = pl.multiple_of(sid * ROW_STRIDE, 8)

        def start_idx(ci, q):
            pltpu.async_copy(e_hbm.at[wid, ci], pbuf.at[q], isems[q])
            pltpu.async_copy(v_hbm.at[wid, ci], vbuf.at[q], isems[q])

        def wait_idx(q):
            pltpu.make_async_copy(e_hbm.at[0, 0], pbuf.at[q],
                                  isems[q]).wait()
            pltpu.make_async_copy(v_hbm.at[0, 0], vbuf.at[q],
                                  isems[q]).wait()

        def start_gather(q, b):
            pltpu.async_copy(h_hbm.at[pbuf.at[q, 0]], rows[b], gsems[b])

        def wait_rows_bytes(sem, b):
            pltpu.make_async_copy(h_hbm.at[pl.ds(0, K)], rows[b],
                                  sem).wait()

        def do_chunk(ci, q, wait_prev_scatter=True, issue_next=True):
            b = q % 2
            qn, bn, q2 = (q + 1) % 4, (b + 1) % 2, (q + 2) % 4
            if issue_next:
                wait_idx(qn)
                start_idx(jnp.minimum(ci + 2, NCHUNK - 1), q2)
                if wait_prev_scatter:
                    wait_rows_bytes(ssems[bn], bn)
                start_gather(qn, bn)
            wait_rows_bytes(gsems[b], b)
            rv = rows[b]

            def scale(gi, c2):
                vvec = vbuf[q, pl.ds(16 * gi, 16)]
                for i in range(16):
                    v = vvec[i]
                    e = 16 * gi + i
                    for j in range(F // 16):
                        sl = pl.ds(16 * j, 16)
                        rv[e, sl] = rv[e, sl] * v
                return c2
            lax.fori_loop(0, K // 16, scale, 0)
            pltpu.async_copy(rv, acc_sh.at[pbuf.at[q, 1]], ssems[b],
                             add=True)

        start_idx(0, 0)
        start_idx(1, 1)
        pltpu.sync_copy(z_hbm.at[pl.ds(rstart, ROW_COPY)],
                        acc_sh.at[pl.ds(rstart, ROW_COPY)])
        plsc.subcore_barrier()
        wait_idx(0)
        start_gather(0, 0)
        do_chunk(0, 0, wait_prev_scatter=False)
        do_chunk(1, 1)
        do_chunk(2, 2)
        do_chunk(3, 3)

        def quad(t, carry):
            ci = 4 * t + 4
            do_chunk(ci, 0)
            do_chunk(ci + 1, 1)
            do_chunk(ci + 2, 2)
            do_chunk(ci + 3, 3)
            return carry
        lax.fori_loop(0, (NCHUNK - 5) // 4, quad, 0)

        wait_idx(1)
        wait_rows_bytes(ssems[1], 1)
        do_chunk(NCHUNK - 1, 0, issue_next=False)
        wait_rows_bytes(ssems[0], 0)

        plsc.subcore_barrier()
        pltpu.sync_copy(acc_sh.at[pl.ds(rstart, ROW_COPY)],
                        out_hbm.at[cid, pl.ds(rstart, ROW_COPY)])

    return k(h, packed, vals3, zeros)


_BM = 1000


def _mm_in(x, w_t, b):
    def body(x_ref, w_ref, b_ref, o_ref):
        o_ref[...] = jnp.dot(x_ref[...], w_ref[...],
                             preferred_element_type=jnp.float32) + b_ref[...]
    return pl.pallas_call(
        body,
        out_shape=jax.ShapeDtypeStruct((N, F), jnp.float32),
    )(x, w_t, b.reshape(1, F))


def _combine_scale(parts, scal):
    def body(s_ref, p_ref, o_ref):
        s = p_ref[0] + p_ref[1]
        o_ref[...] = jnp.where(s > 0, s, (jnp.exp(s) - 1.0)) * s_ref[0]
    return pl.pallas_call(
        body,
        in_specs=[pl.BlockSpec(memory_space=pltpu.SMEM),
                  pl.BlockSpec((NC, N, F), lambda: (0, 0, 0))],
        out_specs=pl.BlockSpec((N, F), lambda: (0, 0)),
        out_shape=jax.ShapeDtypeStruct((N, F), jnp.float32),
    )(scal, parts)


def _combine_mm_out(parts, w_t, b):
    def body(p_ref, w_ref, b_ref, o_ref):
        s = p_ref[0] + p_ref[1]
        h = jnp.where(s > 0, s, (jnp.exp(s) - 1.0))
        o_ref[...] = jnp.dot(h, w_ref[...],
                             preferred_element_type=jnp.float32) + b_ref[...]
    return pl.pallas_call(
        body,
        out_shape=jax.ShapeDtypeStruct((N, F), jnp.float32),
    )(parts, w_t, b.reshape(1, F))


def kernel(x, edge_index, edge_vals, W1, b1, scalar, Wout, bout):
    pad = ((0, 0), (0, EPT - EPW))
    src3 = jnp.pad(edge_index[1].reshape(NW, EPW), pad)
    padblk = (jnp.arange(NW * (EPT - EPW), dtype=jnp.int32)
              .reshape(NW, EPT - EPW) % N)
    dst3 = jnp.concatenate([edge_index[0].reshape(NW, EPW), padblk], axis=1)
    packed = jnp.concatenate([src3.reshape(NW, NCHUNK, 1, K),
                              dst3.reshape(NW, NCHUNK, 1, K)], axis=2)
    vals3 = jnp.pad(edge_vals.reshape(NW, EPW), pad).reshape(NW, NCHUNK, K)
    zeros = jnp.zeros((N, F), jnp.float32)

    h = _mm_in(x, W1.T, b1)
    parts = _spmm_partials(h, packed, vals3, zeros)
    for _ in range(2):
        g = _combine_scale(parts, scalar)
        parts = _spmm_partials(g, packed, vals3, zeros)
    return _combine_mm_out(parts, Wout.T, bout)


# speedup vs baseline: 1.5615x; 1.5615x over previous
import functools

import jax
import jax.numpy as jnp
from jax import lax
from jax.experimental import pallas as pl
from jax.experimental.pallas import tpu as pltpu
from jax.experimental.pallas import tpu_sc as plsc

N = 10000
F = 128
E = 320000

NC = 2
NS = 16
NW = NC * NS
EPW = E // NW
K = 96
NCHUNK = 105
EPT = K * NCHUNK
ROW_STRIDE = 624
ROW_COPY = 640


def _spmm_partials(h, packed, vals3, zeros):
    mesh = plsc.VectorSubcoreMesh(core_axis_name="c", subcore_axis_name="s")

    @functools.partial(
        pl.kernel,
        out_type=jax.ShapeDtypeStruct((NC, N, F), jnp.float32),
        mesh=mesh,
        scratch_types=[
            pltpu.VMEM((4, 2, K), jnp.int32),
            pltpu.VMEM((4, K), jnp.float32),
            pltpu.VMEM((K, F), jnp.float32),
            pltpu.VMEM((K, F), jnp.float32),
            pltpu.VMEM_SHARED((N, F), jnp.float32),
            [pltpu.SemaphoreType.DMA] * 4,
            [pltpu.SemaphoreType.DMA] * 2,
            [pltpu.SemaphoreType.DMA] * 2,
        ],
    )
    def k(h_hbm, e_hbm, v_hbm, z_hbm, out_hbm,
          pbuf, vbuf, rows0_v, rows1_v, acc_sh, isems, gsems, ssems):
        cid = lax.axis_index("c")
        sid = lax.axis_index("s")
        wid = cid * NS + sid
        rows = (rows0_v, rows1_v)

        rstart = pl.multiple_of(sid * ROW_STRIDE, 8)

        def start_idx(ci, q):
            pltpu.async_copy(e_hbm.at[wid, ci], pbuf.at[q], isems[q])
            pltpu.async_copy(v_hbm.at[wid, ci], vbuf.at[q], isems[q])

        def wait_idx(q):
            pltpu.make_async_copy(e_hbm.at[0, 0], pbuf.at[q],
                                  isems[q]).wait()
            pltpu.make_async_copy(v_hbm.at[0, 0], vbuf.at[q],
                                  isems[q]).wait()

        def start_gather(q, b):
            pltpu.async_copy(h_hbm.at[pbuf.at[q, 0]], rows[b], gsems[b])

        def wait_rows_bytes(sem, b):
            pltpu.make_async_copy(h_hbm.at[pl.ds(0, K)], rows[b],
                                  sem).wait()

        def do_chunk(ci, q, wait_prev_scatter=True, issue_next=True):
            b = q % 2
            qn, bn, q2 = (q + 1) % 4, (b + 1) % 2, (q + 2) % 4
            if issue_next:
                wait_idx(qn)
                start_idx(jnp.minimum(ci + 2, NCHUNK - 1), q2)
                if wait_prev_scatter:
                    wait_rows_bytes(ssems[bn], bn)
                start_gather(qn, bn)
            wait_rows_bytes(gsems[b], b)
            rv = rows[b]

            def scale(gi, c2):
                vvec = vbuf[q, pl.ds(16 * gi, 16)]
                for i in range(16):
                    v = vvec[i]
                    e = 16 * gi + i
                    for j in range(F // 16):
                        sl = pl.ds(16 * j, 16)
                        rv[e, sl] = rv[e, sl] * v
                return c2
            lax.fori_loop(0, K // 16, scale, 0)
            pltpu.async_copy(rv, acc_sh.at[pbuf.at[q, 1]], ssems[b],
                             add=True)

        start_idx(0, 0)
        start_idx(1, 1)
        pltpu.sync_copy(z_hbm.at[pl.ds(rstart, ROW_COPY)],
                        acc_sh.at[pl.ds(rstart, ROW_COPY)])
        plsc.subcore_barrier()
        wait_idx(0)
        start_gather(0, 0)
        do_chunk(0, 0, wait_prev_scatter=False)
        do_chunk(1, 1)
        do_chunk(2, 2)
        do_chunk(3, 3)

        def quad(t, carry):
            ci = 4 * t + 4
            do_chunk(ci, 0)
            do_chunk(ci + 1, 1)
            do_chunk(ci + 2, 2)
            do_chunk(ci + 3, 3)
            return carry
        lax.fori_loop(0, (NCHUNK - 5) // 4, quad, 0)

        wait_idx(1)
        wait_rows_bytes(ssems[1], 1)
        do_chunk(NCHUNK - 1, 0, issue_next=False)
        wait_rows_bytes(ssems[0], 0)

        plsc.subcore_barrier()
        pltpu.sync_copy(acc_sh.at[pl.ds(rstart, ROW_COPY)],
                        out_hbm.at[cid, pl.ds(rstart, ROW_COPY)])

    return k(h, packed, vals3, zeros)


_BM = 1000


def _mm_in(x, w_t, b):
    def body(x_ref, w_ref, b_ref, o_ref):
        o_ref[...] = jnp.dot(x_ref[...], w_ref[...],
                             preferred_element_type=jnp.float32) + b_ref[...]
    return pl.pallas_call(
        body,
        out_shape=jax.ShapeDtypeStruct((N, F), jnp.float32),
    )(x, w_t, b.reshape(1, F))


def _combine_scale(parts, scal):
    def body(s_ref, p_ref, o_ref):
        s = p_ref[0] + p_ref[1]
        o_ref[...] = jnp.where(s > 0, s, (jnp.exp(s) - 1.0)) * s_ref[0]
    return pl.pallas_call(
        body,
        in_specs=[pl.BlockSpec(memory_space=pltpu.SMEM),
                  pl.BlockSpec((NC, N, F), lambda: (0, 0, 0))],
        out_specs=pl.BlockSpec((N, F), lambda: (0, 0)),
        out_shape=jax.ShapeDtypeStruct((N, F), jnp.float32),
    )(scal, parts)


def _combine_mm_out(parts, w_t, b):
    def body(p_ref, w_ref, b_ref, o_ref):
        s = p_ref[0] + p_ref[1]
        h = jnp.where(s > 0, s, (jnp.exp(s) - 1.0))
        o_ref[...] = jnp.dot(h, w_ref[...],
                             preferred_element_type=jnp.float32) + b_ref[...]
    return pl.pallas_call(
        body,
        out_shape=jax.ShapeDtypeStruct((N, F), jnp.float32),
    )(parts, w_t, b.reshape(1, F))


def kernel(x, edge_index, edge_vals, W1, b1, scalar, Wout, bout):
    padblk = (jnp.arange(NW * (EPT - EPW), dtype=jnp.int32)
              .reshape(NW, EPT - EPW) % N)
    src3 = jnp.concatenate([edge_index[1].reshape(NW, EPW), padblk], axis=1)
    dst3 = jnp.concatenate([edge_index[0].reshape(NW, EPW), padblk], axis=1)
    packed = jnp.concatenate([src3.reshape(NW, NCHUNK, 1, K),
                              dst3.reshape(NW, NCHUNK, 1, K)], axis=2)
    vals3 = jnp.pad(edge_vals.reshape(NW, EPW),
                    ((0, 0), (0, EPT - EPW))).reshape(NW, NCHUNK, K)
    zeros = jnp.zeros((N, F), jnp.float32)

    h = _mm_in(x, W1.T, b1)
    parts = _spmm_partials(h, packed, vals3, zeros)
    for _ in range(2):
        g = _combine_scale(parts, scalar)
        parts = _spmm_partials(g, packed, vals3, zeros)
    return _combine_mm_out(parts, Wout.T, bout)


# speedup vs baseline: 1.5636x; 1.0014x over previous
import functools

import jax
import jax.numpy as jnp
from jax import lax
from jax.experimental import pallas as pl
from jax.experimental.pallas import tpu as pltpu
from jax.experimental.pallas import tpu_sc as plsc

N = 10000
F = 128
E = 320000

NC = 2
NS = 16
NW = NC * NS
EPW = E // NW
K = 96
NCHUNK = 105
EPT = K * NCHUNK
ROW_STRIDE = 624
ROW_COPY = 640


def _spmm_partials(h, packed, vals3, zeros):
    mesh = plsc.VectorSubcoreMesh(core_axis_name="c", subcore_axis_name="s")

    @functools.partial(
        pl.kernel,
        out_type=jax.ShapeDtypeStruct((NC, N, F), jnp.float32),
        mesh=mesh,
        scratch_types=[
            pltpu.VMEM((4, 2, K), jnp.int32),
            pltpu.VMEM((4, K), jnp.float32),
            pltpu.VMEM((K, F), jnp.float32),
            pltpu.VMEM((K, F), jnp.float32),
            pltpu.VMEM_SHARED((N, F), jnp.float32),
            [pltpu.SemaphoreType.DMA] * 4,
            [pltpu.SemaphoreType.DMA] * 2,
            [pltpu.SemaphoreType.DMA] * 2,
        ],
    )
    def k(h_hbm, e_hbm, v_hbm, z_hbm, out_hbm,
          pbuf, vbuf, rows0_v, rows1_v, acc_sh, isems, gsems, ssems):
        cid = lax.axis_index("c")
        sid = lax.axis_index("s")
        wid = cid * NS + sid
        rows = (rows0_v, rows1_v)

        rstart = pl.multiple_of(sid * ROW_STRIDE, 8)

        def start_idx(ci, q):
            pltpu.async_copy(e_hbm.at[wid, ci], pbuf.at[q], isems[q])
            pltpu.async_copy(v_hbm.at[wid, ci], vbuf.at[q], isems[q])

        def wait_idx(q):
            pltpu.make_async_copy(e_hbm.at[0, 0], pbuf.at[q],
                                  isems[q]).wait()
            pltpu.make_async_copy(v_hbm.at[0, 0], vbuf.at[q],
                                  isems[q]).wait()

        def start_gather(q, b):
            pltpu.async_copy(h_hbm.at[pbuf.at[q, 0]], rows[b], gsems[b])

        def wait_rows_bytes(sem, b):
            pltpu.make_async_copy(h_hbm.at[pl.ds(0, K)], rows[b],
                                  sem).wait()

        def do_chunk(ci, q, wait_prev_scatter=True, issue_next=True):
            b = q % 2
            qn, bn, q2 = (q + 1) % 4, (b + 1) % 2, (q + 2) % 4
            if issue_next:
                wait_idx(qn)
                start_idx(jnp.minimum(ci + 2, NCHUNK - 1), q2)
                if wait_prev_scatter:
                    wait_rows_bytes(ssems[bn], bn)
                start_gather(qn, bn)
            wait_rows_bytes(gsems[b], b)
            rv = rows[b]

            def scale(gi, c2):
                vvec = vbuf[q, pl.ds(16 * gi, 16)]
                for i in range(16):
                    v = vvec[i]
                    e = 16 * gi + i
                    for j in range(F // 16):
                        sl = pl.ds(16 * j, 16)
                        rv[e, sl] = rv[e, sl] * v
                return c2
            lax.fori_loop(0, K // 16, scale, 0)
            pltpu.async_copy(rv, acc_sh.at[pbuf.at[q, 1]], ssems[b],
                             add=True)

        start_idx(0, 0)
        start_idx(1, 1)
        pltpu.sync_copy(z_hbm.at[pl.ds(rstart, ROW_COPY)],
                        acc_sh.at[pl.ds(rstart, ROW_COPY)])
        wait_idx(0)
        start_gather(0, 0)
        plsc.subcore_barrier()
        do_chunk(0, 0, wait_prev_scatter=False)
        do_chunk(1, 1)
        do_chunk(2, 2)
        do_chunk(3, 3)

        def quad(t, carry):
            ci = 4 * t + 4
            do_chunk(ci, 0)
            do_chunk(ci + 1, 1)
            do_chunk(ci + 2, 2)
            do_chunk(ci + 3, 3)
            return carry
        lax.fori_loop(0, (NCHUNK - 5) // 4, quad, 0)

        wait_idx(1)
        wait_rows_bytes(ssems[1], 1)
        do_chunk(NCHUNK - 1, 0, issue_next=False)
        wait_rows_bytes(ssems[0], 0)

        plsc.subcore_barrier()
        pltpu.sync_copy(acc_sh.at[pl.ds(rstart, ROW_COPY)],
                        out_hbm.at[cid, pl.ds(rstart, ROW_COPY)])

    return k(h, packed, vals3, zeros)


_BM = 1000


def _mm_in(x, w_t, b):
    def body(x_ref, w_ref, b_ref, o_ref):
        o_ref[...] = jnp.dot(x_ref[...], w_ref[...],
                             preferred_element_type=jnp.float32) + b_ref[...]
    return pl.pallas_call(
        body,
        out_shape=jax.ShapeDtypeStruct((N, F), jnp.float32),
    )(x, w_t, b.reshape(1, F))


def _combine_scale(parts, scal):
    def body(s_ref, p_ref, o_ref):
        s = p_ref[0] + p_ref[1]
        o_ref[...] = jnp.where(s > 0, s, (jnp.exp(s) - 1.0)) * s_ref[0]
    return pl.pallas_call(
        body,
        in_specs=[pl.BlockSpec(memory_space=pltpu.SMEM),
                  pl.BlockSpec((NC, N, F), lambda: (0, 0, 0))],
        out_specs=pl.BlockSpec((N, F), lambda: (0, 0)),
        out_shape=jax.ShapeDtypeStruct((N, F), jnp.float32),
    )(scal, parts)


def _combine_mm_out(parts, w_t, b):
    def body(p_ref, w_ref, b_ref, o_ref):
        s = p_ref[0] + p_ref[1]
        h = jnp.where(s > 0, s, (jnp.exp(s) - 1.0))
        o_ref[...] = jnp.dot(h, w_ref[...],
                             preferred_element_type=jnp.float32) + b_ref[...]
    return pl.pallas_call(
        body,
        out_shape=jax.ShapeDtypeStruct((N, F), jnp.float32),
    )(parts, w_t, b.reshape(1, F))


def kernel(x, edge_index, edge_vals, W1, b1, scalar, Wout, bout):
    padblk = (jnp.arange(NW * (EPT - EPW), dtype=jnp.int32)
              .reshape(NW, EPT - EPW) % N)
    src3 = jnp.concatenate([edge_index[1].reshape(NW, EPW), padblk], axis=1)
    dst3 = jnp.concatenate([edge_index[0].reshape(NW, EPW), padblk], axis=1)
    packed = jnp.concatenate([src3.reshape(NW, NCHUNK, 1, K),
                              dst3.reshape(NW, NCHUNK, 1, K)], axis=2)
    vals3 = jnp.pad(edge_vals.reshape(NW, EPW),
                    ((0, 0), (0, EPT - EPW))).reshape(NW, NCHUNK, K)
    zeros = jnp.zeros((N, F), jnp.float32)

    h = _mm_in(x, W1.T, b1)
    parts = _spmm_partials(h, packed, vals3, zeros)
    for _ in range(2):
        g = _combine_scale(parts, scalar)
        parts = _spmm_partials(g, packed, vals3, zeros)
    return _combine_mm_out(parts, Wout.T, bout)
